# Initial kernel scaffold; baseline (speedup 1.0000x reference)
import jax
import jax.numpy as jnp
from jax.experimental import pallas as pl


def kernel(pred, target):
    raise NotImplementedError("write your pallas kernel here")


# speedup vs baseline: 15.3057x; 15.3057x over previous
import jax
import jax.numpy as jnp
from jax.experimental import pallas as pl
from jax.experimental.pallas import tpu as pltpu

_C = 10
_IGN = _C - 1
_B, _H, _W = 16, 512, 512
_BH = 128


def _tc_body(pred_ref, tgt_ref, nll_ref, cnt_ref, pres_ref):
    b = pl.program_id(0)
    h = pl.program_id(1)

    @pl.when((b == 0) & (h == 0))
    def _():
        nll_ref[0, 0] = 0.0
        cnt_ref[0, 0] = 0.0
        pres_ref[0, 0] = 0

    t = tgt_ref[0]
    s = jnp.zeros(t.shape, jnp.float32)
    sel = jnp.zeros(t.shape, jnp.float32)
    pres = jnp.int32(0)
    for c in range(_C):
        x = pred_ref[0, c]
        s = s + jnp.exp(x)
        mc = t == c
        sel = jnp.where(mc, x, sel)
        pres = pres | jnp.where(jnp.any(mc), jnp.int32(1 << c), jnp.int32(0))
    maskf = (t != _IGN).astype(jnp.float32)
    nll = (jnp.log(s) - sel) * maskf
    nll_ref[0, 0] += jnp.sum(nll)
    cnt_ref[0, 0] += jnp.sum(maskf)
    pres_ref[0, 0] = pres_ref[0, 0] | pres


def _tc_call(pred, target):
    grid = (_B, _H // _BH)
    return pl.pallas_call(
        _tc_body,
        grid=grid,
        in_specs=[
            pl.BlockSpec((1, _C, _BH, _W), lambda b, h: (b, 0, h, 0)),
            pl.BlockSpec((1, _BH, _W), lambda b, h: (b, h, 0)),
        ],
        out_specs=[
            pl.BlockSpec((1, 1), lambda b, h: (0, 0), memory_space=pltpu.SMEM),
            pl.BlockSpec((1, 1), lambda b, h: (0, 0), memory_space=pltpu.SMEM),
            pl.BlockSpec((1, 1), lambda b, h: (0, 0), memory_space=pltpu.SMEM),
        ],
        out_shape=[
            jax.ShapeDtypeStruct((1, 1), jnp.float32),
            jax.ShapeDtypeStruct((1, 1), jnp.float32),
            jax.ShapeDtypeStruct((1, 1), jnp.int32),
        ],
    )(pred, target)


def kernel(pred, target):
    nll, cnt, pres = _tc_call(pred, target)
    bits = (pres[0, 0] >> jnp.arange(_C, dtype=jnp.int32)) & 1
    n_unique = jnp.sum(bits)
    loss = nll[0, 0] / cnt[0, 0]
    return jnp.where(n_unique < 2, 0.0 * loss, loss)
